# baseline (speedup 1.0000x reference)
import functools

import jax
import jax.numpy as jnp
from jax import lax
from jax.experimental import pallas as pl
from jax.experimental.pallas import tpu as pltpu
from jax.experimental.pallas import tpu_sc as plsc

_N = 1048576
_K = 64
_ALPHA = 0.1
_NC = 2
_NS = 16
_NW = _NC * _NS
_L = 16
_C = _N // _NW


_NCHUNK = 32
_S = _C // _NCHUNK


def _sc_body(dist_hbm, assign_hbm, sums_hbm, hist_hbm,
             a_v, lut_v, idx0_v, idx1_v, vals0_v, vals1_v, hist_v, out16_v,
             sem0, sem1):
    cid = lax.axis_index("c")
    sid = lax.axis_index("s")
    wid = sid * _NC + cid
    base = wid * _C

    pltpu.sync_copy(assign_hbm.at[pl.ds(base, _C)], a_v)

    iota16 = lax.iota(jnp.int32, _L)
    ones16 = jnp.ones((_L,), jnp.float32)
    bufs = ((idx0_v, vals0_v, sem0), (idx1_v, vals1_v, sem1))

    zero16 = jnp.zeros((_L,), jnp.float32)
    for j in range(_K // _L):
        hist_v[pl.ds(j * _L, _L)] = zero16
        k_vec = j * _L + iota16
        lut_v[pl.ds(j * _L, _L)] = ((k_vec >> 3) << 23) + ((k_vec & 7) << 7)

    def build_chunk(t):
        idx_v, vals_v, sem = bufs[t % 2]
        blk0 = (base + t * _S) >> 7

        def build_block(b, carry):
            ip = ((blk0 + b) << 10) + iota16
            off = b << 7
            for j2 in range(128 // _L):
                a_sl = a_v[pl.ds(t * _S + off + j2 * _L, _L)]
                ap = plsc.load_gather(lut_v, [a_sl])
                idx_v[pl.ds(off + j2 * _L, _L)] = ap + ip
                plsc.addupdate_scatter(hist_v, [a_sl], ones16)
                ip = ip + _L
            return carry

        lax.fori_loop(0, _S // 128, build_block, 0)
        return pltpu.async_copy(dist_hbm.at[idx_v], vals_v, sem)

    def accum_chunk(t, acc):
        _, vals_v, _ = bufs[t % 2]

        def accum(j, a):
            a0, a1 = a
            off = j << 7
            for j2 in range(0, 128 // _L, 2):
                a0 = a0 + vals_v[pl.ds(off + j2 * _L, _L)]
                a1 = a1 + vals_v[pl.ds(off + (j2 + 1) * _L, _L)]
            return (a0, a1)

        return lax.fori_loop(0, _S // 128, accum, acc)

    acc = (jnp.zeros((_L,), jnp.float32), jnp.zeros((_L,), jnp.float32))
    copies = [None, None]
    copies[0] = build_chunk(0)
    for t in range(1, _NCHUNK):
        copies[t % 2] = build_chunk(t)
        copies[(t - 1) % 2].wait()
        acc = accum_chunk(t - 1, acc)
    copies[(_NCHUNK - 1) % 2].wait()
    acc = accum_chunk(_NCHUNK - 1, acc)

    out16_v[...] = acc[0] + acc[1]
    pltpu.sync_copy(out16_v, sums_hbm.at[wid])
    pltpu.sync_copy(hist_v, hist_hbm.at[wid])


_sc_call = pl.kernel(
    _sc_body,
    out_type=[
        jax.ShapeDtypeStruct((_NW, _L), jnp.float32),
        jax.ShapeDtypeStruct((_NW, _K), jnp.float32),
    ],
    mesh=plsc.VectorSubcoreMesh(
        core_axis_name="c", subcore_axis_name="s",
        num_cores=_NC, num_subcores=_NS,
    ),
    compiler_params=pltpu.CompilerParams(needs_layout_passes=False),
    scratch_types=[
        pltpu.VMEM((_C,), jnp.int32),
        pltpu.VMEM((_K,), jnp.int32),
        pltpu.VMEM((_S,), jnp.int32),
        pltpu.VMEM((_S,), jnp.int32),
        pltpu.VMEM((_S,), jnp.float32),
        pltpu.VMEM((_S,), jnp.float32),
        pltpu.VMEM((_K,), jnp.float32),
        pltpu.VMEM((_L,), jnp.float32),
        pltpu.SemaphoreType.DMA,
        pltpu.SemaphoreType.DMA,
    ],
)


def _tc_body(sums_ref, hist_ref, out_ref):
    total = jnp.sum(sums_ref[...])
    counts = jnp.sum(hist_ref[...], axis=0)
    probs = counts * (1.0 / _N)
    entropy = -jnp.sum(probs * jnp.log(probs + 1e-8))
    out_ref[0, 0] = total * (1.0 / _N) - _ALPHA * entropy


_tc_call = pl.pallas_call(
    _tc_body,
    out_shape=jax.ShapeDtypeStruct((1, 1), jnp.float32),
    out_specs=pl.BlockSpec(memory_space=pltpu.SMEM),
)


@jax.jit
def _impl(distances, assignments):
    dist_phys = (
        distances.reshape(8192, 128, 8, 8).transpose(2, 0, 3, 1).reshape(-1)
    )
    sums, hists = _sc_call(dist_phys, assignments.astype(jnp.int32))
    return _tc_call(sums, hists)[0, 0]


def kernel(distances, assignments):
    return _impl(distances, assignments)


# speedup vs baseline: 1.1601x; 1.1601x over previous
import functools

import jax
import jax.numpy as jnp
from jax import lax
from jax.experimental import pallas as pl
from jax.experimental.pallas import tpu as pltpu
from jax.experimental.pallas import tpu_sc as plsc

_N = 1048576
_K = 64
_ALPHA = 0.1
_NC = 2
_NS = 16
_NW = _NC * _NS
_L = 16
_C = _N // _NW


_NCHUNK = 8
_S = _C // _NCHUNK


def _sc_body(dist_hbm, assign_hbm, sums_hbm, hist_hbm,
             a_v, lut_v, idx0_v, idx1_v, vals0_v, vals1_v, hist_v, out16_v,
             sem0, sem1):
    cid = lax.axis_index("c")
    sid = lax.axis_index("s")
    wid = sid * _NC + cid
    base = wid * _C

    pltpu.sync_copy(assign_hbm.at[pl.ds(base, _C)], a_v)

    iota16 = lax.iota(jnp.int32, _L)
    ones16 = jnp.ones((_L,), jnp.float32)
    bufs = ((idx0_v, vals0_v, sem0), (idx1_v, vals1_v, sem1))

    zero16 = jnp.zeros((_L,), jnp.float32)
    for j in range(_K // _L):
        hist_v[pl.ds(j * _L, _L)] = zero16
        k_vec = j * _L + iota16
        lut_v[pl.ds(j * _L, _L)] = ((k_vec >> 3) << 23) + ((k_vec & 7) << 7)

    def build_chunk(t):
        idx_v, vals_v, sem = bufs[t % 2]
        blk0 = (base + t * _S) >> 7

        def build_block(b, carry):
            ip = ((blk0 + b) << 10) + iota16
            off = b << 7
            for j2 in range(128 // _L):
                a_sl = a_v[pl.ds(t * _S + off + j2 * _L, _L)]
                ap = plsc.load_gather(lut_v, [a_sl])
                idx_v[pl.ds(off + j2 * _L, _L)] = ap + ip
                plsc.addupdate_scatter(hist_v, [a_sl], ones16)
                ip = ip + _L
            return carry

        lax.fori_loop(0, _S // 128, build_block, 0)
        return pltpu.async_copy(dist_hbm.at[idx_v], vals_v, sem)

    def accum_chunk(t, acc):
        _, vals_v, _ = bufs[t % 2]

        def accum(j, a):
            a0, a1 = a
            off = j << 7
            for j2 in range(0, 128 // _L, 2):
                a0 = a0 + vals_v[pl.ds(off + j2 * _L, _L)]
                a1 = a1 + vals_v[pl.ds(off + (j2 + 1) * _L, _L)]
            return (a0, a1)

        return lax.fori_loop(0, _S // 128, accum, acc)

    acc = (jnp.zeros((_L,), jnp.float32), jnp.zeros((_L,), jnp.float32))
    copies = [None, None]
    copies[0] = build_chunk(0)
    for t in range(1, _NCHUNK):
        copies[t % 2] = build_chunk(t)
        copies[(t - 1) % 2].wait()
        acc = accum_chunk(t - 1, acc)
    copies[(_NCHUNK - 1) % 2].wait()
    acc = accum_chunk(_NCHUNK - 1, acc)

    out16_v[...] = acc[0] + acc[1]
    pltpu.sync_copy(out16_v, sums_hbm.at[wid])
    pltpu.sync_copy(hist_v, hist_hbm.at[wid])


_sc_call = pl.kernel(
    _sc_body,
    out_type=[
        jax.ShapeDtypeStruct((_NW, _L), jnp.float32),
        jax.ShapeDtypeStruct((_NW, _K), jnp.float32),
    ],
    mesh=plsc.VectorSubcoreMesh(
        core_axis_name="c", subcore_axis_name="s",
        num_cores=_NC, num_subcores=_NS,
    ),
    compiler_params=pltpu.CompilerParams(needs_layout_passes=False),
    scratch_types=[
        pltpu.VMEM((_C,), jnp.int32),
        pltpu.VMEM((_K,), jnp.int32),
        pltpu.VMEM((_S,), jnp.int32),
        pltpu.VMEM((_S,), jnp.int32),
        pltpu.VMEM((_S,), jnp.float32),
        pltpu.VMEM((_S,), jnp.float32),
        pltpu.VMEM((_K,), jnp.float32),
        pltpu.VMEM((_L,), jnp.float32),
        pltpu.SemaphoreType.DMA,
        pltpu.SemaphoreType.DMA,
    ],
)


def _tc_body(sums_ref, hist_ref, out_ref):
    total = jnp.sum(sums_ref[...])
    counts = jnp.sum(hist_ref[...], axis=0)
    probs = counts * (1.0 / _N)
    entropy = -jnp.sum(probs * jnp.log(probs + 1e-8))
    out_ref[0, 0] = total * (1.0 / _N) - _ALPHA * entropy


_tc_call = pl.pallas_call(
    _tc_body,
    out_shape=jax.ShapeDtypeStruct((1, 1), jnp.float32),
    out_specs=pl.BlockSpec(memory_space=pltpu.SMEM),
)


@jax.jit
def _impl(distances, assignments):
    dist_phys = (
        distances.reshape(8192, 128, 8, 8).transpose(2, 0, 3, 1).reshape(-1)
    )
    sums, hists = _sc_call(dist_phys, assignments.astype(jnp.int32))
    return _tc_call(sums, hists)[0, 0]


def kernel(distances, assignments):
    return _impl(distances, assignments)
